# baseline (speedup 1.0000x reference)
import jax
import jax.numpy as jnp
from jax.experimental import pallas as pl
from jax.experimental.pallas import tpu as pltpu

EPS = 1e-5

RB = 192
TPB = 3
N_ROWS = 4608
N_CODES = 8192
D = 64


def _body(u_ref, es_ref, x_ref, out_ref, ebf_ref, e2_ref, colf_ref):
    i = pl.program_id(0)

    @pl.when(i == 0)
    def _():
        emb = es_ref[...] / jnp.maximum(u_ref[...], EPS)
        ebf_ref[...] = emb.astype(jnp.bfloat16)
        e2_ref[...] = jnp.sum(emb * emb, axis=1)[None, :]
        colf_ref[...] = jax.lax.broadcasted_iota(
            jnp.int32, (1, N_CODES), 1).astype(jnp.float32)

    m = i % TPB
    xfull = x_ref[0]
    xs = jnp.where(
        m == 0, xfull[:, 0:RB],
        jnp.where(m == 1, xfull[:, RB:2 * RB], xfull[:, 2 * RB:3 * RB]))
    xt = jnp.transpose(xs)
    x2 = jnp.sum(xt * xt, axis=1, keepdims=True)
    xbf = (xt * -2.0).astype(jnp.bfloat16)

    s = jax.lax.dot_general(
        xbf, ebf_ref[...],
        dimension_numbers=(((1,), (1,)), ((), ())),
        preferred_element_type=jnp.float32,
    )
    d2 = (x2 + e2_ref[...]) + s

    lmin = jnp.min(d2, axis=1, keepdims=True)
    lpos = jnp.maximum(lmin, 0.0)
    sv = jnp.sqrt(lpos)
    li = jax.lax.bitcast_convert_type(lpos, jnp.int32)
    bnd = lpos
    for k in range(1, 6):
        ck = jax.lax.bitcast_convert_type(li + k, jnp.float32)
        bnd = jnp.where(jnp.sqrt(ck) <= sv, ck, bnd)
    bnd = jnp.where(lmin > 0.0, bnd, 0.0)

    lidx = jnp.min(jnp.where(d2 <= bnd, colf_ref[...], jnp.float32(1e30)),
                   axis=1, keepdims=True)
    out_ref[...] = lidx.astype(jnp.int32)


def kernel(x, cluster_usage, embedding_sum):
    B, _, T = x.shape
    usage = cluster_usage.reshape(N_CODES, 1)

    codes = pl.pallas_call(
        _body,
        grid=(N_ROWS // RB,),
        in_specs=[
            pl.BlockSpec((N_CODES, 1), lambda i: (0, 0)),
            pl.BlockSpec((N_CODES, D), lambda i: (0, 0)),
            pl.BlockSpec((1, D, T), lambda i: (i // TPB, 0, 0)),
        ],
        out_specs=pl.BlockSpec((RB, 1), lambda i: (i, 0)),
        out_shape=jax.ShapeDtypeStruct((N_ROWS, 1), jnp.int32),
        scratch_shapes=[
            pltpu.VMEM((N_CODES, D), jnp.bfloat16),
            pltpu.VMEM((1, N_CODES), jnp.float32),
            pltpu.VMEM((1, N_CODES), jnp.float32),
        ],
        compiler_params=pltpu.CompilerParams(
            dimension_semantics=("arbitrary",)),
    )(usage, embedding_sum, x)

    return codes.reshape(B, 1, T)


# speedup vs baseline: 1.1266x; 1.1266x over previous
import jax
import jax.numpy as jnp
from jax.experimental import pallas as pl
from jax.experimental.pallas import tpu as pltpu

EPS = 1e-5

N_ROWS = 4608
N_CODES = 8192
D = 64


def _body(u_ref, es_ref, x_ref, out_ref, ebf_ref, e2_ref, rowf_ref):
    i = pl.program_id(0)

    @pl.when(i == 0)
    def _():
        emb = es_ref[...] / jnp.maximum(u_ref[...], EPS)
        ebf_ref[...] = emb.astype(jnp.bfloat16)
        e2_ref[...] = jnp.sum(emb * emb, axis=1, keepdims=True)
        rowf_ref[...] = jax.lax.broadcasted_iota(
            jnp.int32, (N_CODES, 1), 0).astype(jnp.float32)

    xfull = x_ref[0]
    x2 = jnp.sum(xfull * xfull, axis=0, keepdims=True)
    xbf = (xfull * -2.0).astype(jnp.bfloat16)

    s = jax.lax.dot_general(
        ebf_ref[...], xbf,
        dimension_numbers=(((1,), (0,)), ((), ())),
        preferred_element_type=jnp.float32,
    )
    d2 = (x2 + e2_ref[...]) + s

    lmin = jnp.min(d2, axis=0, keepdims=True)
    lpos = jnp.maximum(lmin, 0.0)
    sv = jnp.sqrt(lpos)
    li = jax.lax.bitcast_convert_type(lpos, jnp.int32)
    bnd = lpos
    for k in range(1, 6):
        ck = jax.lax.bitcast_convert_type(li + k, jnp.float32)
        bnd = jnp.where(jnp.sqrt(ck) <= sv, ck, bnd)
    bnd = jnp.where(lmin > 0.0, bnd, 0.0)

    lidx = jnp.min(jnp.where(d2 <= bnd, rowf_ref[...], jnp.float32(1e30)),
                   axis=0, keepdims=True)
    out_ref[...] = lidx.astype(jnp.int32)[None]


def kernel(x, cluster_usage, embedding_sum):
    B, _, T = x.shape
    usage = cluster_usage.reshape(N_CODES, 1)

    codes = pl.pallas_call(
        _body,
        grid=(B,),
        in_specs=[
            pl.BlockSpec((N_CODES, 1), lambda i: (0, 0)),
            pl.BlockSpec((N_CODES, D), lambda i: (0, 0)),
            pl.BlockSpec((1, D, T), lambda i: (i, 0, 0)),
        ],
        out_specs=pl.BlockSpec((1, 1, T), lambda i: (i, 0, 0)),
        out_shape=jax.ShapeDtypeStruct((B, 1, T), jnp.int32),
        scratch_shapes=[
            pltpu.VMEM((N_CODES, D), jnp.bfloat16),
            pltpu.VMEM((N_CODES, 1), jnp.float32),
            pltpu.VMEM((N_CODES, 1), jnp.float32),
        ],
        compiler_params=pltpu.CompilerParams(
            dimension_semantics=("arbitrary",)),
    )(usage, embedding_sum, x)

    return codes
